# baseline (speedup 1.0000x reference)
import functools

import jax
import jax.numpy as jnp
from jax import lax
from jax.experimental import pallas as pl
from jax.experimental.pallas import tpu as pltpu
from jax.experimental.pallas import tpu_sc as plsc

B, H, W = 16, 512, 512
L = 16
ROW_MIN = W // 128
CHUNK_ROWS = 64
N_CHUNKS = (H // 2) // CHUNK_ROWS
PROB_T = 0.5
NPTS_T = 10
LOOSE = 64

_mesh = plsc.VectorSubcoreMesh(core_axis_name="c", subcore_axis_name="s")


_SC_SCRATCH = [
    pltpu.VMEM((CHUNK_ROWS * ROW_MIN, 128), jnp.float32),
    pltpu.VMEM((W,), jnp.float32),
    pltpu.VMEM((L,), jnp.int32),
    pltpu.VMEM((L,), jnp.int32),
    pltpu.VMEM((L,), jnp.int32),
]


def _sc_body(mask_hbm, bbox_hbm, part_hbm, chunk, colbuf, meta, pmeta, bstage):
    c = lax.axis_index("c")
    s = lax.axis_index("s")
    b = c * 8 + s // 2
    h = s % 2
    r0 = h * (H // 2)

    io = lax.broadcasted_iota(jnp.int32, (L,), 0)
    one_i = jnp.ones((L,), jnp.int32)
    zero_i = jnp.zeros((L,), jnp.int32)

    for j in range(W // L):
        colbuf[pl.ds(j * L, L)] = jnp.full((L,), -1.0, jnp.float32)

    cntv = jnp.zeros((L,), jnp.int32)
    rmin_s = jnp.int32(H + 1)
    rmax_s = jnp.int32(-1)

    for ch in range(N_CHUNKS):
        start = (r0 + ch * CHUNK_ROWS) * ROW_MIN
        pltpu.sync_copy(mask_hbm.at[b, pl.ds(start, CHUNK_ROWS * ROW_MIN)],
                        chunk)

        def row_body(r, carry, _ch=ch):
            cv, rmn, rmx = carry
            rowacc = jnp.full((L,), -1.0, jnp.float32)
            for cr in range(ROW_MIN):
                for k in range(128 // L):
                    j = cr * (128 // L) + k
                    v = chunk[r * ROW_MIN + cr, pl.ds(k * L, L)]
                    rowacc = jnp.maximum(rowacc, v)
                    cv = cv + jnp.where(v > PROB_T, one_i, zero_i)
                    ca = colbuf[pl.ds(j * L, L)]
                    colbuf[pl.ds(j * L, L)] = jnp.maximum(ca, v)
            rany = jnp.max(rowacc) > PROB_T
            rr = r0 + _ch * CHUNK_ROWS + r
            rmn = jnp.where(rany, jnp.minimum(rmn, rr), rmn)
            rmx = jnp.where(rany, jnp.maximum(rmx, rr), rmx)
            return cv, rmn, rmx

        cntv, rmin_s, rmax_s = lax.fori_loop(
            0, CHUNK_ROWS, row_body, (cntv, rmin_s, rmax_s))

    cnt_mine = jnp.sum(cntv)

    cminv = jnp.full((L,), W + 1, jnp.int32)
    cmaxv = jnp.full((L,), -1, jnp.int32)
    for j in range(W // L):
        cc = colbuf[pl.ds(j * L, L)]
        m = cc > PROB_T
        idx = io + j * L
        cminv = jnp.minimum(cminv, jnp.where(m, idx, W + 1))
        cmaxv = jnp.maximum(cmaxv, jnp.where(m, idx, -1))
    cmin_s = jnp.min(cminv)
    cmax_s = jnp.max(cmaxv)

    wid = c * 16 + s
    mv = jnp.where(io == 0, cnt_mine,
                   jnp.where(io == 1, rmin_s,
                             jnp.where(io == 2, rmax_s,
                                       jnp.where(io == 3, cmin_s,
                                                 jnp.where(io == 4, cmax_s, 0)))))
    meta[...] = mv
    pltpu.sync_copy(meta, part_hbm.at[wid])
    plsc.subcore_barrier()

    pltpu.sync_copy(part_hbm.at[wid ^ 1], pmeta)
    pm = pmeta[...]
    cnt_tot = cnt_mine + pm[0]
    rmin = jnp.minimum(rmin_s, pm[1])
    rmax = jnp.maximum(rmax_s, pm[2])
    cmin = jnp.minimum(cmin_s, pm[3])
    cmax = jnp.maximum(cmax_s, pm[4])

    valid = cnt_tot >= NPTS_T
    rmin_f = jnp.where(valid, jnp.maximum(rmin - LOOSE, 0), 0)
    rmax_f = jnp.where(valid, jnp.minimum(rmax + LOOSE, H - 1), H - 1)
    cmin_f = jnp.where(valid, jnp.maximum(cmin - LOOSE, 0), 0)
    cmax_f = jnp.where(valid, jnp.minimum(cmax + LOOSE, W - 1), W - 1)

    bvec = jnp.where(io == 0, rmin_f,
                     jnp.where(io == 1, cmin_f,
                               jnp.where(io == 2, rmax_f,
                                         jnp.where(io == 3, cmax_f, 0))))

    @pl.when(h == 0)
    def _():
        bstage[...] = bvec
        pltpu.sync_copy(bstage, bbox_hbm.at[b])


_sc_bbox = pl.kernel(
    _sc_body,
    mesh=_mesh,
    out_type=[jax.ShapeDtypeStruct((B, 16), jnp.int32),
              jax.ShapeDtypeStruct((32, L), jnp.int32)],
    compiler_params=pltpu.CompilerParams(needs_layout_passes=False),
    scratch_types=_SC_SCRATCH,
)


def _paint_body(bbox_ref, out_ref):
    b = pl.program_id(0)
    rmin = bbox_ref[b, 0]
    cmin = bbox_ref[b, 1]
    rmax = bbox_ref[b, 2]
    cmax = bbox_ref[b, 3]
    rr = lax.broadcasted_iota(jnp.int32, (1, H, W), 1)
    cc = lax.broadcasted_iota(jnp.int32, (1, H, W), 2)
    att = (rr >= rmin) & (rr <= rmax) & (cc >= cmin) & (cc <= cmax)
    out_ref[...] = att.astype(jnp.float32)


_paint = pl.pallas_call(
    _paint_body,
    grid=(B,),
    in_specs=[pl.BlockSpec(memory_space=pltpu.SMEM)],
    out_specs=pl.BlockSpec((1, H, W), lambda b: (b, 0, 0)),
    out_shape=jax.ShapeDtypeStruct((B, H, W), jnp.float32),
)


def kernel(mask):
    mask_r = mask.reshape(B, (H * W) // 128, 128)
    bbox_wide, _ = _sc_bbox(mask_r)
    att = _paint(bbox_wide)
    return att, bbox_wide[:, :4]


# speedup vs baseline: 1.0909x; 1.0909x over previous
import functools

import jax
import jax.numpy as jnp
from jax import lax
from jax.experimental import pallas as pl
from jax.experimental.pallas import tpu as pltpu
from jax.experimental.pallas import tpu_sc as plsc

B, H, W = 16, 512, 512
L = 16
ROW_MIN = W // 128
CHUNK_ROWS = 64
N_CHUNKS = (H // 2) // CHUNK_ROWS
PROB_T = 0.5
NPTS_T = 10
LOOSE = 64

_mesh = plsc.VectorSubcoreMesh(core_axis_name="c", subcore_axis_name="s")


_SC_SCRATCH = [
    pltpu.VMEM((2, CHUNK_ROWS * ROW_MIN, 128), jnp.float32),
    pltpu.VMEM((L,), jnp.int32),
    pltpu.VMEM((L,), jnp.int32),
    pltpu.VMEM((L,), jnp.int32),
    pltpu.SemaphoreType.DMA,
    pltpu.SemaphoreType.DMA,
]

_NJ = W // L


def _sc_body(mask_hbm, bbox_hbm, part_hbm, chunk, meta, pmeta, bstage,
             sem_a, sem_b):
    c = lax.axis_index("c")
    s = lax.axis_index("s")
    b = c * 8 + s // 2
    h = s % 2
    r0 = h * (H // 2)

    io = lax.broadcasted_iota(jnp.int32, (L,), 0)
    one_i = jnp.ones((L,), jnp.int32)

    sems = (sem_a, sem_b)

    def src(ch):
        start = (r0 + ch * CHUNK_ROWS) * ROW_MIN
        return mask_hbm.at[b, pl.ds(start, CHUNK_ROWS * ROW_MIN)]

    carry = (
        [jnp.zeros((L,), jnp.int32) for _ in range(4)]
        + [jnp.full((L,), H + 1, jnp.int32),
           jnp.full((L,), -1, jnp.int32),
           jnp.zeros((L,), jnp.int32)]
        + [jnp.full((L,), -1.0, jnp.float32) for _ in range(_NJ)]
    )

    handles = [pltpu.async_copy(src(0), chunk.at[0], sem_a), None]
    for ch in range(N_CHUNKS):
        buf = ch % 2
        if ch + 1 < N_CHUNKS:
            handles[1 - buf] = pltpu.async_copy(
                src(ch + 1), chunk.at[1 - buf], sems[1 - buf])
        handles[buf].wait()

        def row_body(r, cr_, _buf=buf):
            cv0, cv1, cv2, cv3, rminv, rmaxv, rsplat = cr_[:7]
            cols = list(cr_[7:])
            cvs = [cv0, cv1, cv2, cv3]
            ras = [None, None, None, None]
            for cr in range(ROW_MIN):
                for k in range(128 // L):
                    j = cr * (128 // L) + k
                    v = chunk[_buf, r * ROW_MIN + cr, pl.ds(k * L, L)]
                    cols[j] = jnp.maximum(cols[j], v)
                    m = v > PROB_T
                    cvs[j % 4] = cvs[j % 4] + plsc.all_reduce_population_count(m)
                    q = j % 4
                    ras[q] = v if ras[q] is None else jnp.maximum(ras[q], v)
            rowm = jnp.maximum(jnp.maximum(ras[0], ras[1]),
                               jnp.maximum(ras[2], ras[3]))
            manyv = rowm > PROB_T
            rminv = jnp.where(manyv, jnp.minimum(rminv, rsplat), rminv)
            rmaxv = jnp.where(manyv, jnp.maximum(rmaxv, rsplat), rmaxv)
            rsplat = rsplat + one_i
            return tuple(cvs) + (rminv, rmaxv, rsplat) + tuple(cols)

        carry = list(lax.fori_loop(0, CHUNK_ROWS, row_body, tuple(carry)))

    cv0, cv1, cv2, cv3, rminv, rmaxv, _ = carry[:7]
    cols = carry[7:]

    cnt_mine = jnp.max(cv0 + cv1 + cv2 + cv3)

    mn = jnp.min(rminv)
    mx = jnp.max(rmaxv)
    rmin_s = jnp.where(mn > H, jnp.int32(H + 1), mn + r0)
    rmax_s = jnp.where(mx < 0, jnp.int32(-1), mx + r0)

    cminv = jnp.full((L,), W + 1, jnp.int32)
    cmaxv = jnp.full((L,), -1, jnp.int32)
    for j in range(_NJ):
        m = cols[j] > PROB_T
        idx = io + j * L
        cminv = jnp.minimum(cminv, jnp.where(m, idx, W + 1))
        cmaxv = jnp.maximum(cmaxv, jnp.where(m, idx, -1))
    cmin_s = jnp.min(cminv)
    cmax_s = jnp.max(cmaxv)

    wid = c * 16 + s
    mv = jnp.where(io == 0, cnt_mine,
                   jnp.where(io == 1, rmin_s,
                             jnp.where(io == 2, rmax_s,
                                       jnp.where(io == 3, cmin_s,
                                                 jnp.where(io == 4, cmax_s, 0)))))
    meta[...] = mv
    pltpu.sync_copy(meta, part_hbm.at[wid])
    plsc.subcore_barrier()

    pltpu.sync_copy(part_hbm.at[wid ^ 1], pmeta)
    pm = pmeta[...]
    cnt_tot = cnt_mine + pm[0]
    rmin = jnp.minimum(rmin_s, pm[1])
    rmax = jnp.maximum(rmax_s, pm[2])
    cmin = jnp.minimum(cmin_s, pm[3])
    cmax = jnp.maximum(cmax_s, pm[4])

    valid = cnt_tot >= NPTS_T
    rmin_f = jnp.where(valid, jnp.maximum(rmin - LOOSE, 0), 0)
    rmax_f = jnp.where(valid, jnp.minimum(rmax + LOOSE, H - 1), H - 1)
    cmin_f = jnp.where(valid, jnp.maximum(cmin - LOOSE, 0), 0)
    cmax_f = jnp.where(valid, jnp.minimum(cmax + LOOSE, W - 1), W - 1)

    bvec = jnp.where(io == 0, rmin_f,
                     jnp.where(io == 1, cmin_f,
                               jnp.where(io == 2, rmax_f,
                                         jnp.where(io == 3, cmax_f, 0))))

    @pl.when(h == 0)
    def _():
        bstage[...] = bvec
        pltpu.sync_copy(bstage, bbox_hbm.at[b])


_sc_bbox = pl.kernel(
    _sc_body,
    mesh=_mesh,
    out_type=[jax.ShapeDtypeStruct((B, 16), jnp.int32),
              jax.ShapeDtypeStruct((32, L), jnp.int32)],
    compiler_params=pltpu.CompilerParams(needs_layout_passes=False),
    scratch_types=_SC_SCRATCH,
)


def _paint_body(bbox_ref, out_ref):
    b = pl.program_id(0)
    rmin = bbox_ref[b, 0]
    cmin = bbox_ref[b, 1]
    rmax = bbox_ref[b, 2]
    cmax = bbox_ref[b, 3]
    rr = lax.broadcasted_iota(jnp.int32, (1, H, W), 1)
    cc = lax.broadcasted_iota(jnp.int32, (1, H, W), 2)
    att = (rr >= rmin) & (rr <= rmax) & (cc >= cmin) & (cc <= cmax)
    out_ref[...] = att.astype(jnp.float32)


_paint = pl.pallas_call(
    _paint_body,
    grid=(B,),
    in_specs=[pl.BlockSpec(memory_space=pltpu.SMEM)],
    out_specs=pl.BlockSpec((1, H, W), lambda b: (b, 0, 0)),
    out_shape=jax.ShapeDtypeStruct((B, H, W), jnp.float32),
)


def kernel(mask):
    mask_r = mask.reshape(B, (H * W) // 128, 128)
    bbox_wide, _ = _sc_bbox(mask_r)
    att = _paint(bbox_wide)
    return att, bbox_wide[:, :4]


# speedup vs baseline: 1.6584x; 1.5202x over previous
import functools

import jax
import jax.numpy as jnp
from jax import lax
from jax.experimental import pallas as pl
from jax.experimental.pallas import tpu as pltpu
from jax.experimental.pallas import tpu_sc as plsc

B, H, W = 16, 512, 512
L = 16
ROW_MIN = W // 128
CHUNK_ROWS = 64
N_CHUNKS = (H // 2) // CHUNK_ROWS
PROB_T = 0.5
NPTS_T = 10
LOOSE = 64

_mesh = plsc.VectorSubcoreMesh(core_axis_name="c", subcore_axis_name="s")


_SC_SCRATCH = [
    pltpu.VMEM((2, CHUNK_ROWS, W), jnp.float32),
    pltpu.VMEM((L,), jnp.int32),
    pltpu.VMEM((L,), jnp.int32),
    pltpu.VMEM((L,), jnp.int32),
    pltpu.SemaphoreType.DMA,
    pltpu.SemaphoreType.DMA,
]

_NJ = W // L
_HJ = _NJ // 2
_RPI = 16


def _sc_body(mask_hbm, bbox_hbm, part_hbm, chunk, meta, pmeta, bstage,
             sem_a, sem_b):
    c = lax.axis_index("c")
    s = lax.axis_index("s")
    b = c * 8 + s // 2
    h = s % 2
    r0 = h * (H // 2)

    io = lax.broadcasted_iota(jnp.int32, (L,), 0)
    one_i = jnp.ones((L,), jnp.int32)
    zero_i = jnp.zeros((L,), jnp.int32)

    sems = (sem_a, sem_b)

    def src(ch):
        return mask_hbm.at[b, pl.ds(r0 + ch * CHUNK_ROWS, CHUNK_ROWS)]

    colmax = [jnp.full((L,), -1.0, jnp.float32) for _ in range(_NJ)]
    rminv = jnp.full((L,), H + 1, jnp.int32)
    rmaxv = jnp.full((L,), -1, jnp.int32)
    cv0 = jnp.zeros((L,), jnp.int32)
    cv1 = jnp.zeros((L,), jnp.int32)

    handles = [pltpu.async_copy(src(0), chunk.at[0], sem_a), None]
    for ch in range(N_CHUNKS):
        buf = ch % 2
        if ch + 1 < N_CHUNKS:
            handles[1 - buf] = pltpu.async_copy(
                src(ch + 1), chunk.at[1 - buf], sems[1 - buf])
        handles[buf].wait()

        for half in range(2):
            def row_body(r2, cr_, _buf=buf, _half=half):
                rminv_, rmaxv_, rsplat, cva, cvb = cr_[:5]
                cols = list(cr_[5:])
                for rr in range(_RPI):
                    ra = rb = None
                    for k in range(_HJ):
                        j = _half * _HJ + k
                        v = chunk[_buf, r2 * _RPI + rr, pl.ds(j * L, L)]
                        cols[k] = jnp.maximum(cols[k], v)
                        pc = plsc.all_reduce_population_count(v > PROB_T)
                        if k % 2 == 0:
                            cva = cva + pc
                            ra = v if ra is None else jnp.maximum(ra, v)
                        else:
                            cvb = cvb + pc
                            rb = v if rb is None else jnp.maximum(rb, v)
                    manyv = jnp.maximum(ra, rb) > PROB_T
                    rminv_ = jnp.where(manyv, jnp.minimum(rminv_, rsplat),
                                       rminv_)
                    rmaxv_ = jnp.where(manyv, jnp.maximum(rmaxv_, rsplat),
                                       rmaxv_)
                    rsplat = rsplat + one_i
                return (rminv_, rmaxv_, rsplat, cva, cvb) + tuple(cols)

            base = jnp.full((L,), ch * CHUNK_ROWS, jnp.int32)
            out = lax.fori_loop(
                0, CHUNK_ROWS // _RPI, row_body,
                (rminv, rmaxv, base, cv0, cv1)
                + tuple(colmax[half * _HJ:(half + 1) * _HJ]))
            rminv, rmaxv, _, cv0, cv1 = out[:5]
            colmax[half * _HJ:(half + 1) * _HJ] = list(out[5:])

    cnt_mine = jnp.max(cv0 + cv1)

    mn = jnp.min(rminv)
    mx = jnp.max(rmaxv)
    rmin_s = jnp.where(mn > H, jnp.int32(H + 1), mn + r0)
    rmax_s = jnp.where(mx < 0, jnp.int32(-1), mx + r0)

    cminv = jnp.full((L,), W + 1, jnp.int32)
    cmaxv = jnp.full((L,), -1, jnp.int32)
    for j in range(_NJ):
        m = colmax[j] > PROB_T
        idx = io + j * L
        cminv = jnp.minimum(cminv, jnp.where(m, idx, W + 1))
        cmaxv = jnp.maximum(cmaxv, jnp.where(m, idx, -1))
    cmin_s = jnp.min(cminv)
    cmax_s = jnp.max(cmaxv)

    wid = c * 16 + s
    mv = jnp.where(io == 0, cnt_mine,
                   jnp.where(io == 1, rmin_s,
                             jnp.where(io == 2, rmax_s,
                                       jnp.where(io == 3, cmin_s,
                                                 jnp.where(io == 4, cmax_s, 0)))))
    meta[...] = mv
    pltpu.sync_copy(meta, part_hbm.at[wid])
    plsc.subcore_barrier()

    pltpu.sync_copy(part_hbm.at[wid ^ 1], pmeta)
    pm = pmeta[...]
    cnt_tot = cnt_mine + pm[0]
    rmin = jnp.minimum(rmin_s, pm[1])
    rmax = jnp.maximum(rmax_s, pm[2])
    cmin = jnp.minimum(cmin_s, pm[3])
    cmax = jnp.maximum(cmax_s, pm[4])

    valid = cnt_tot >= NPTS_T
    rmin_f = jnp.where(valid, jnp.maximum(rmin - LOOSE, 0), 0)
    rmax_f = jnp.where(valid, jnp.minimum(rmax + LOOSE, H - 1), H - 1)
    cmin_f = jnp.where(valid, jnp.maximum(cmin - LOOSE, 0), 0)
    cmax_f = jnp.where(valid, jnp.minimum(cmax + LOOSE, W - 1), W - 1)

    bvec = jnp.where(io == 0, rmin_f,
                     jnp.where(io == 1, cmin_f,
                               jnp.where(io == 2, rmax_f,
                                         jnp.where(io == 3, cmax_f, 0))))

    @pl.when(h == 0)
    def _():
        bstage[...] = bvec
        pltpu.sync_copy(bstage, bbox_hbm.at[b])


_sc_bbox = pl.kernel(
    _sc_body,
    mesh=_mesh,
    out_type=[jax.ShapeDtypeStruct((B, 16), jnp.int32),
              jax.ShapeDtypeStruct((32, L), jnp.int32)],
    compiler_params=pltpu.CompilerParams(needs_layout_passes=False),
    scratch_types=_SC_SCRATCH,
)


def _paint_body(bbox_ref, out_ref):
    b = pl.program_id(0)
    rmin = bbox_ref[b, 0]
    cmin = bbox_ref[b, 1]
    rmax = bbox_ref[b, 2]
    cmax = bbox_ref[b, 3]
    rr = lax.broadcasted_iota(jnp.int32, (1, H, W), 1)
    cc = lax.broadcasted_iota(jnp.int32, (1, H, W), 2)
    att = (rr >= rmin) & (rr <= rmax) & (cc >= cmin) & (cc <= cmax)
    out_ref[...] = att.astype(jnp.float32)


_paint = pl.pallas_call(
    _paint_body,
    grid=(B,),
    in_specs=[pl.BlockSpec(memory_space=pltpu.SMEM)],
    out_specs=pl.BlockSpec((1, H, W), lambda b: (b, 0, 0)),
    out_shape=jax.ShapeDtypeStruct((B, H, W), jnp.float32),
)


def kernel(mask):
    bbox_wide, _ = _sc_bbox(mask)
    att = _paint(bbox_wide)
    return att, bbox_wide[:, :4]
